# Initial kernel scaffold; baseline (speedup 1.0000x reference)
import functools

import jax
import jax.numpy as jnp
from jax import lax
from jax.experimental import pallas as pl
from jax.experimental.pallas import tpu as pltpu
from jax.experimental.pallas import tpu_sc as plsc

_N = 10000
_E = 320000
_D = 128

_NC = 2
_NS = 16
_NW = _NC * _NS
_EPW = _E // _NW
_B = 80
_CHUNKS = _EPW // _B
_RPT = _N // _NS


def _make_sc_agg(with_deg):
  mesh = plsc.VectorSubcoreMesh(core_axis_name="c", subcore_axis_name="s")
  out_type = [jax.ShapeDtypeStruct((_NC, _N, _D), jnp.float32)]
  scratch = [
      pltpu.VMEM((_CHUNKS, _B), jnp.int32),
      pltpu.VMEM((_CHUNKS, _B), jnp.int32),
      pltpu.VMEM((_B, _D), jnp.float32),
      pltpu.VMEM_SHARED((_N, _D), jnp.float32),
      pltpu.SemaphoreType.DMA,
  ]
  if with_deg:
    out_type.append(jax.ShapeDtypeStruct((_NC, _N, 16), jnp.float32))
    scratch.insert(3, pltpu.VMEM((_B, 16), jnp.float32))
    scratch.insert(4, pltpu.VMEM_SHARED((_N, 16), jnp.float32))

  def body(*refs):
    if with_deg:
      (h_hbm, row3, col3, z_hbm, z16_hbm, ones_hbm,
       agg_out, deg_out,
       rowv, colv, buf, onesv, agg_sh, deg_sh, sem) = refs
    else:
      (h_hbm, row3, col3, z_hbm,
       agg_out,
       rowv, colv, buf, agg_sh, sem) = refs

    c = lax.axis_index("c")
    s = lax.axis_index("s")
    wid = c * _NS + s

    pltpu.sync_copy(z_hbm.at[pl.ds(s * _RPT, _RPT)],
                    agg_sh.at[pl.ds(s * _RPT, _RPT)])
    if with_deg:
      pltpu.sync_copy(z16_hbm.at[pl.ds(s * _RPT, _RPT)],
                      deg_sh.at[pl.ds(s * _RPT, _RPT)])
      pltpu.sync_copy(ones_hbm, onesv)
    pltpu.sync_copy(row3.at[wid], rowv)
    pltpu.sync_copy(col3.at[wid], colv)
    plsc.subcore_barrier()

    def step(j, carry):
      pltpu.async_copy(h_hbm.at[rowv.at[j]], buf, sem).wait()
      pltpu.sync_copy(buf, agg_sh.at[colv.at[j]], add=True)
      if with_deg:
        pltpu.sync_copy(onesv, deg_sh.at[colv.at[j]], add=True)
      return carry

    lax.fori_loop(0, _CHUNKS, step, 0)

    plsc.subcore_barrier()
    pltpu.sync_copy(agg_sh.at[pl.ds(s * _RPT, _RPT)],
                    agg_out.at[c, pl.ds(s * _RPT, _RPT)])
    if with_deg:
      pltpu.sync_copy(deg_sh.at[pl.ds(s * _RPT, _RPT)],
                      deg_out.at[c, pl.ds(s * _RPT, _RPT)])

  return pl.kernel(body, out_type=out_type, mesh=mesh, scratch_types=scratch)


_sc_agg_deg = _make_sc_agg(True)
_sc_agg = _make_sc_agg(False)


def _combine_body(with_relu, p_ref, h_ref, dp_ref, w_ref, b_ref, o_ref):
  agg = p_ref[0] + p_ref[1] + h_ref[...]
  deg = dp_ref[0, :, 0:1] + dp_ref[1, :, 0:1] + 1.0
  neigh = agg / deg
  y = jnp.dot(neigh, w_ref[...], preferred_element_type=jnp.float32,
              precision=lax.Precision.HIGHEST) + b_ref[...]
  if with_relu:
    y = jnp.maximum(y, 0.0)
  o_ref[...] = y


def _combine(p, h, dp, w, b, with_relu):
  rows = 2000
  grid = _N // rows
  return pl.pallas_call(
      functools.partial(_combine_body, with_relu),
      grid=(grid,),
      in_specs=[
          pl.BlockSpec((_NC, rows, _D), lambda i: (0, i, 0)),
          pl.BlockSpec((rows, _D), lambda i: (i, 0)),
          pl.BlockSpec((_NC, rows, 16), lambda i: (0, i, 0)),
          pl.BlockSpec((_D, _D), lambda i: (0, 0)),
          pl.BlockSpec((1, _D), lambda i: (0, 0)),
      ],
      out_specs=pl.BlockSpec((rows, _D), lambda i: (i, 0)),
      out_shape=jax.ShapeDtypeStruct((_N, _D), jnp.float32),
  )(p, h, dp, w, b)


@jax.jit
def kernel(x, edge_index, W1, b1, W2, b2):
  row3 = edge_index[0].reshape(_NW, _CHUNKS, _B)
  col3 = edge_index[1].reshape(_NW, _CHUNKS, _B)
  z = jnp.zeros((_N, _D), jnp.float32)
  z16 = jnp.zeros((_N, 16), jnp.float32)
  ones = jnp.ones((_B, 16), jnp.float32)
  b1r = b1.reshape(1, _D)
  b2r = b2.reshape(1, _D)

  p1, dp = _sc_agg_deg(x, row3, col3, z, z16, ones)
  h1 = _combine(p1, x, dp, W1, b1r, with_relu=True)
  (p2,) = _sc_agg(h1, row3, col3, z)
  out = _combine(p2, h1, dp, W2, b2r, with_relu=False)
  return out


# speedup vs baseline: 6.2584x; 6.2584x over previous
import functools

import jax
import jax.numpy as jnp
from jax import lax
from jax.experimental import pallas as pl
from jax.experimental.pallas import tpu as pltpu
from jax.experimental.pallas import tpu_sc as plsc

_N = 10000
_NP = 10240
_E = 320000
_D = 128

_NC = 2
_NS = 16
_NW = _NC * _NS
_EPW = _E // _NW
_B = 80
_CHUNKS = _EPW // _B
_RPT = _NP // _NS

_mesh = plsc.VectorSubcoreMesh(core_axis_name="c", subcore_axis_name="s")


@functools.partial(
    pl.kernel,
    out_type=jax.ShapeDtypeStruct((_NC, _NP, _D), jnp.float32),
    mesh=_mesh,
    scratch_types=[
        pltpu.VMEM((_CHUNKS, _B), jnp.int32),
        pltpu.VMEM((_CHUNKS, _B), jnp.int32),
        pltpu.VMEM((_B, _D), jnp.float32),
        pltpu.VMEM_SHARED((_NP, _D), jnp.float32),
        pltpu.SemaphoreType.DMA,
    ],
)
def _sc_agg(h_hbm, row3, col3, z_hbm, agg_out, rowv, colv, buf, agg_sh, sem):
  c = lax.axis_index("c")
  s = lax.axis_index("s")
  wid = c * _NS + s

  pltpu.sync_copy(z_hbm.at[pl.ds(s * _RPT, _RPT)],
                  agg_sh.at[pl.ds(s * _RPT, _RPT)])
  pltpu.sync_copy(row3.at[wid], rowv)
  pltpu.sync_copy(col3.at[wid], colv)
  plsc.subcore_barrier()

  def step(j, carry):
    pltpu.async_copy(h_hbm.at[rowv.at[j]], buf, sem).wait()
    pltpu.sync_copy(buf, agg_sh.at[colv.at[j]], add=True)
    return carry

  lax.fori_loop(0, _CHUNKS, step, 0)

  plsc.subcore_barrier()
  pltpu.sync_copy(agg_sh.at[pl.ds(s * _RPT, _RPT)],
                  agg_out.at[c, pl.ds(s * _RPT, _RPT)])


@functools.partial(
    pl.kernel,
    out_type=jax.ShapeDtypeStruct((_NC, _NP, _D), jnp.float32),
    mesh=_mesh,
    scratch_types=[
        pltpu.VMEM((_CHUNKS, _B), jnp.int32),
        pltpu.VMEM((_B, _D), jnp.float32),
        pltpu.VMEM_SHARED((_NP, _D), jnp.float32),
    ],
)
def _sc_deg(col3, z16_hbm, ones_hbm, deg_out, colv, onesv, deg_sh):
  c = lax.axis_index("c")
  s = lax.axis_index("s")
  wid = c * _NS + s

  pltpu.sync_copy(z16_hbm.at[pl.ds(s * _RPT, _RPT)],
                  deg_sh.at[pl.ds(s * _RPT, _RPT)])
  pltpu.sync_copy(ones_hbm, onesv)
  pltpu.sync_copy(col3.at[wid], colv)
  plsc.subcore_barrier()

  def step(j, carry):
    pltpu.sync_copy(onesv, deg_sh.at[colv.at[j]], add=True)
    return carry

  lax.fori_loop(0, _CHUNKS, step, 0)

  plsc.subcore_barrier()
  pltpu.sync_copy(deg_sh.at[pl.ds(s * _RPT, _RPT)],
                  deg_out.at[c, pl.ds(s * _RPT, _RPT)])


def _combine_body(with_relu, p_ref, h_ref, dp_ref, w_ref, b_ref, o_ref):
  agg = p_ref[0] + p_ref[1] + h_ref[...]
  deg = dp_ref[0, :, 0:1] + dp_ref[1, :, 0:1] + 1.0
  neigh = agg / deg
  y = jnp.dot(neigh, w_ref[...], preferred_element_type=jnp.float32,
              precision=lax.Precision.HIGHEST) + b_ref[...]
  if with_relu:
    y = jnp.maximum(y, 0.0)
  o_ref[...] = y


def _combine(p, h, dp, w, b, with_relu):
  rows = 2048
  grid = _NP // rows
  return pl.pallas_call(
      functools.partial(_combine_body, with_relu),
      grid=(grid,),
      in_specs=[
          pl.BlockSpec((_NC, rows, _D), lambda i: (0, i, 0)),
          pl.BlockSpec((rows, _D), lambda i: (i, 0)),
          pl.BlockSpec((_NC, rows, _D), lambda i: (0, i, 0)),
          pl.BlockSpec((_D, _D), lambda i: (0, 0)),
          pl.BlockSpec((1, _D), lambda i: (0, 0)),
      ],
      out_specs=pl.BlockSpec((rows, _D), lambda i: (i, 0)),
      out_shape=jax.ShapeDtypeStruct((_NP, _D), jnp.float32),
  )(p, h, dp, w, b)


@jax.jit
def kernel(x, edge_index, W1, b1, W2, b2):
  row3 = edge_index[0].reshape(_NW, _CHUNKS, _B)
  col3 = edge_index[1].reshape(_NW, _CHUNKS, _B)
  xp = jnp.pad(x, ((0, _NP - _N), (0, 0)))
  z = jnp.zeros((_NP, _D), jnp.float32)
  ones = jnp.ones((_B, _D), jnp.float32)
  b1r = b1.reshape(1, _D)
  b2r = b2.reshape(1, _D)

  dp = _sc_deg(col3, z, ones)
  p1 = _sc_agg(xp, row3, col3, z)
  h1 = _combine(p1, xp, dp, W1, b1r, with_relu=True)
  p2 = _sc_agg(h1, row3, col3, z)
  out = _combine(p2, h1, dp, W2, b2r, with_relu=False)
  return out[:_N]


# speedup vs baseline: 7.7379x; 1.2364x over previous
import functools

import jax
import jax.numpy as jnp
from jax import lax
from jax.experimental import pallas as pl
from jax.experimental.pallas import tpu as pltpu
from jax.experimental.pallas import tpu_sc as plsc

_N = 10000
_NP = 10240
_E = 320000
_D = 128

_NC = 2
_NS = 16
_NW = _NC * _NS
_EPW = _E // _NW
_B = 80
_CHUNKS = _EPW // _B
_RPT = _NP // _NS

_mesh = plsc.VectorSubcoreMesh(core_axis_name="c", subcore_axis_name="s")


@functools.partial(
    pl.kernel,
    out_type=jax.ShapeDtypeStruct((_NC, _NP, _D), jnp.float32),
    mesh=_mesh,
    scratch_types=[
        pltpu.VMEM((_EPW,), jnp.int32),
        pltpu.VMEM((_CHUNKS, _B), jnp.int32),
        pltpu.VMEM_SHARED((_NP, _D), jnp.float32),
        pltpu.VMEM((_B, _D), jnp.float32),
        pltpu.VMEM((_B, _D), jnp.float32),
        pltpu.SemaphoreType.DMA,
        pltpu.SemaphoreType.DMA,
        pltpu.SemaphoreType.DMA,
        pltpu.SemaphoreType.DMA,
    ],
)
def _sc_agg(h_hbm, row2, col3, z_hbm, agg_out, rowv, colv, agg_sh,
            buf0, buf1, gs0, gs1, ss0, ss1):
  bufs = (buf0, buf1)
  gsem = (gs0, gs1)
  ssem = (ss0, ss1)
  c = lax.axis_index("c")
  s = lax.axis_index("s")
  wid = c * _NS + s

  pltpu.sync_copy(z_hbm.at[pl.ds(s * _RPT, _RPT)],
                  agg_sh.at[pl.ds(s * _RPT, _RPT)])
  pltpu.sync_copy(row2.at[wid], rowv)
  pltpu.sync_copy(col3.at[wid], colv)
  plsc.subcore_barrier()

  def gather(j, b):
    pltpu.async_copy(h_hbm.at[rowv.at[pl.ds(j * _B, _B)]], bufs[b], gsem[b])

  def gather_wait(j, b):
    pltpu.make_async_copy(h_hbm.at[rowv.at[pl.ds(j * _B, _B)]], bufs[b],
                          gsem[b]).wait()

  def scatter(j, b):
    pltpu.async_copy(bufs[b], agg_sh.at[colv.at[j]], ssem[b], add=True)

  def scatter_wait(j, b):
    pltpu.make_async_copy(bufs[b], agg_sh.at[colv.at[j]], ssem[b]).wait()

  gather(0, 0)
  gather(1, 1)

  def group(g, carry):
    j0 = 2 * g
    j1 = j0 + 1
    gather_wait(j0, 0)
    scatter(j0, 0)
    gather_wait(j1, 1)
    scatter(j1, 1)
    scatter_wait(j0, 0)
    gather(j0 + 2, 0)
    scatter_wait(j1, 1)
    gather(j1 + 2, 1)
    return carry

  lax.fori_loop(0, 61, group, 0)

  gather_wait(122, 0)
  scatter(122, 0)
  gather_wait(123, 1)
  scatter(123, 1)
  scatter_wait(122, 0)
  gather(124, 0)
  scatter_wait(123, 1)
  gather_wait(124, 0)
  scatter(124, 0)
  scatter_wait(124, 0)

  plsc.subcore_barrier()
  pltpu.sync_copy(agg_sh.at[pl.ds(s * _RPT, _RPT)],
                  agg_out.at[c, pl.ds(s * _RPT, _RPT)])


@functools.partial(
    pl.kernel,
    out_type=jax.ShapeDtypeStruct((_NC, _NP, _D), jnp.float32),
    mesh=_mesh,
    scratch_types=[
        pltpu.VMEM((_CHUNKS, _B), jnp.int32),
        pltpu.VMEM((_B, _D), jnp.float32),
        pltpu.VMEM_SHARED((_NP, _D), jnp.float32),
        pltpu.SemaphoreType.DMA,
    ],
)
def _sc_deg(col3, z_hbm, ones_hbm, deg_out, colv, onesv, deg_sh, sem):
  c = lax.axis_index("c")
  s = lax.axis_index("s")
  wid = c * _NS + s

  pltpu.sync_copy(z_hbm.at[pl.ds(s * _RPT, _RPT)],
                  deg_sh.at[pl.ds(s * _RPT, _RPT)])
  pltpu.sync_copy(ones_hbm, onesv)
  pltpu.sync_copy(col3.at[wid], colv)
  plsc.subcore_barrier()

  def wave(w, carry):
    def fire(j, carry2):
      pltpu.async_copy(onesv, deg_sh.at[colv.at[w * 25 + j]], sem, add=True)
      return carry2

    lax.fori_loop(0, 25, fire, 0)

    def drain(j, carry2):
      pltpu.make_async_copy(onesv, deg_sh.at[colv.at[w * 25 + j]], sem).wait()
      return carry2

    lax.fori_loop(0, 25, drain, 0)
    return carry

  lax.fori_loop(0, _CHUNKS // 25, wave, 0)

  plsc.subcore_barrier()
  pltpu.sync_copy(deg_sh.at[pl.ds(s * _RPT, _RPT)],
                  deg_out.at[c, pl.ds(s * _RPT, _RPT)])


def _combine_body(with_relu, p_ref, h_ref, dp_ref, w_ref, b_ref, o_ref):
  agg = p_ref[0] + p_ref[1] + h_ref[...]
  deg = dp_ref[0, :, 0:1] + dp_ref[1, :, 0:1] + 1.0
  neigh = agg / deg
  y = jnp.dot(neigh, w_ref[...], preferred_element_type=jnp.float32,
              precision=lax.Precision.HIGHEST) + b_ref[...]
  if with_relu:
    y = jnp.maximum(y, 0.0)
  o_ref[...] = y


def _combine(p, h, dp, w, b, with_relu):
  rows = 2048
  grid = _NP // rows
  return pl.pallas_call(
      functools.partial(_combine_body, with_relu),
      grid=(grid,),
      in_specs=[
          pl.BlockSpec((_NC, rows, _D), lambda i: (0, i, 0)),
          pl.BlockSpec((rows, _D), lambda i: (i, 0)),
          pl.BlockSpec((_NC, rows, _D), lambda i: (0, i, 0)),
          pl.BlockSpec((_D, _D), lambda i: (0, 0)),
          pl.BlockSpec((1, _D), lambda i: (0, 0)),
      ],
      out_specs=pl.BlockSpec((rows, _D), lambda i: (i, 0)),
      out_shape=jax.ShapeDtypeStruct((_NP, _D), jnp.float32),
  )(p, h, dp, w, b)


@jax.jit
def kernel(x, edge_index, W1, b1, W2, b2):
  row2 = edge_index[0].reshape(_NW, _EPW)
  col3 = edge_index[1].reshape(_NW, _CHUNKS, _B)
  xp = jnp.pad(x, ((0, _NP - _N), (0, 0)))
  z = jnp.zeros((_NP, _D), jnp.float32)
  ones = jnp.ones((_B, _D), jnp.float32)
  b1r = b1.reshape(1, _D)
  b2r = b2.reshape(1, _D)

  dp = _sc_deg(col3, z, ones)
  p1 = _sc_agg(xp, row2, col3, z)
  h1 = _combine(p1, xp, dp, W1, b1r, with_relu=True)
  p2 = _sc_agg(h1, row2, col3, z)
  out = _combine(p2, h1, dp, W2, b2r, with_relu=False)
  return out[:_N]
